# Initial kernel scaffold; baseline (speedup 1.0000x reference)
import functools

import jax
import jax.numpy as jnp
from jax import lax
from jax.experimental import pallas as pl
from jax.experimental.pallas import tpu as pltpu
from jax.experimental.pallas import tpu_sc as plsc

B, L, RANK, HIDDEN = 4096, 200, 32, 128
N = B * L
NC, NS = 2, 16
NW = NC * NS
TOK_W = N // NW
KSTR = 5
CH = KSTR * 128
GB = 64
TOK_BLK = GB * L
GRP = TOK_BLK // 4
NP = N // 4
BLK_W = TOK_W // TOK_BLK
CH_G = GRP // CH

_mesh = plsc.VectorSubcoreMesh(
    core_axis_name="c", subcore_axis_name="s", num_cores=NC, num_subcores=NS
)


@functools.partial(
    pl.kernel,
    out_type=jax.ShapeDtypeStruct((NP, HIDDEN), jnp.float32),
    mesh=_mesh,
    scratch_types=[
        pltpu.VMEM((KSTR, 128), jnp.int32),
        pltpu.VMEM((CH, RANK), jnp.float32),
        pltpu.SemaphoreType.DMA,
    ],
)
def _sc_gather(ids_hbm, table_hbm, p_hbm, idx_v, rows_v, sem):
    c = lax.axis_index("c")
    s = lax.axis_index("s")
    wid = s * NC + c
    base = wid * TOK_W
    for gg in range(BLK_W):
        for q in range(4):
            def step(i, _, gg=gg, q=q):
                tok0 = base + gg * TOK_BLK + q * GRP + i * CH
                pltpu.sync_copy(ids_hbm.at[pl.ds(tok0 // 128, KSTR)], idx_v)
                cps = [
                    pltpu.async_copy(
                        table_hbm.at[idx_v.at[j]],
                        rows_v.at[pl.ds(j * 128, 128)],
                        sem,
                    )
                    for j in range(KSTR)
                ]
                for cp in cps:
                    cp.wait()
                row0 = (wid * BLK_W + gg) * GRP + i * CH
                pltpu.sync_copy(
                    rows_v,
                    p_hbm.at[pl.ds(row0, CH), pl.ds(32 * q, 32)],
                )
                return 0
            lax.fori_loop(0, CH_G, step, 0)


def _tc_body(p_ref, w2_ref, posb_ref, out_ref):
    p = p_ref[...]
    w2 = w2_ref[...]
    pos = jnp.broadcast_to(
        posb_ref[...][None], (GRP // L, L, HIDDEN)
    ).reshape(GRP, HIDDEN)
    for q in range(4):
        xq = p[:, 32 * q:32 * (q + 1)]
        yq = lax.dot_general(
            xq, w2, (((1,), (1,)), ((), ())),
            preferred_element_type=jnp.float32,
        )
        out_ref[pl.ds(q * GRP, GRP), :] = yq + pos


_tc_project = pl.pallas_call(
    _tc_body,
    grid=(N // TOK_BLK,),
    in_specs=[
        pl.BlockSpec((GRP, HIDDEN), lambda g: (g, 0)),
        pl.BlockSpec((HIDDEN, RANK), lambda g: (0, 0)),
        pl.BlockSpec((L, HIDDEN), lambda g: (0, 0)),
    ],
    out_specs=pl.BlockSpec((TOK_BLK, HIDDEN), lambda g: (g, 0)),
    out_shape=jax.ShapeDtypeStruct((N, HIDDEN), jnp.float32),
)


def kernel(input_ids, E1, W2, b2, pos_emb):
    ids = input_ids.reshape(N // 128, 128).astype(jnp.int32)
    p = _sc_gather(ids, E1)
    posb = pos_emb[0, :L] + b2[None, :]
    out2 = _tc_project(p, W2, posb)
    return out2.reshape(B, L, HIDDEN)


# speedup vs baseline: 26.3254x; 26.3254x over previous
import functools

import jax
import jax.numpy as jnp
from jax import lax
from jax.experimental import pallas as pl
from jax.experimental.pallas import tpu as pltpu
from jax.experimental.pallas import tpu_sc as plsc

B, L, RANK, HIDDEN = 4096, 200, 32, 128
N = B * L
NC, NS = 2, 16
NW = NC * NS
TOK_W = N // NW
KSTR = 5
CH = KSTR * 128
GB = 64
TOK_BLK = GB * L
GRP = TOK_BLK // 4
NP = N // 4
BLK_W = TOK_W // TOK_BLK
CH_G = GRP // CH

_mesh = plsc.VectorSubcoreMesh(
    core_axis_name="c", subcore_axis_name="s", num_cores=NC, num_subcores=NS
)


@functools.partial(
    pl.kernel,
    out_type=jax.ShapeDtypeStruct((NP, HIDDEN), jnp.float32),
    mesh=_mesh,
    scratch_types=[
        pltpu.VMEM((KSTR, 128), jnp.int32),
        pltpu.VMEM((CH, RANK), jnp.float32),
        pltpu.SemaphoreType.DMA,
    ],
    compiler_params=pltpu.CompilerParams(use_tc_tiling_on_sc=False),
)
def _sc_gather(ids_hbm, table_hbm, p_hbm, idx_v, rows_v, sem):
    c = lax.axis_index("c")
    s = lax.axis_index("s")
    wid = s * NC + c
    base = wid * TOK_W
    for gg in range(BLK_W):
        for q in range(4):
            def step(i, _, gg=gg, q=q):
                tok0 = base + gg * TOK_BLK + q * GRP + i * CH
                pltpu.sync_copy(ids_hbm.at[pl.ds(tok0 // 128, KSTR)], idx_v)
                cps = [
                    pltpu.async_copy(
                        table_hbm.at[idx_v.at[j]],
                        rows_v.at[pl.ds(j * 128, 128)],
                        sem,
                    )
                    for j in range(KSTR)
                ]
                for cp in cps:
                    cp.wait()
                row0 = (wid * BLK_W + gg) * GRP + i * CH
                pltpu.sync_copy(
                    rows_v,
                    p_hbm.at[pl.ds(row0, CH), pl.ds(32 * q, 32)],
                )
                return 0
            lax.fori_loop(0, CH_G, step, 0)


def _tc_body(p_ref, w2_ref, posb_ref, out_ref):
    p = p_ref[...]
    w2 = w2_ref[...]
    pos = jnp.broadcast_to(
        posb_ref[...][None], (GRP // L, L, HIDDEN)
    ).reshape(GRP, HIDDEN)
    for q in range(4):
        xq = p[:, 32 * q:32 * (q + 1)]
        yq = lax.dot_general(
            xq, w2, (((1,), (1,)), ((), ())),
            preferred_element_type=jnp.float32,
        )
        out_ref[pl.ds(q * GRP, GRP), :] = yq + pos


_tc_project = pl.pallas_call(
    _tc_body,
    grid=(N // TOK_BLK,),
    in_specs=[
        pl.BlockSpec((GRP, HIDDEN), lambda g: (g, 0)),
        pl.BlockSpec((HIDDEN, RANK), lambda g: (0, 0)),
        pl.BlockSpec((L, HIDDEN), lambda g: (0, 0)),
    ],
    out_specs=pl.BlockSpec((TOK_BLK, HIDDEN), lambda g: (g, 0)),
    out_shape=jax.ShapeDtypeStruct((N, HIDDEN), jnp.float32),
)


def kernel(input_ids, E1, W2, b2, pos_emb):
    ids = input_ids.reshape(N // 128, 128).astype(jnp.int32)
    p = _sc_gather(ids, E1)
    posb = pos_emb[0, :L] + b2[None, :]
    out2 = _tc_project(p, W2, posb)
    return out2.reshape(B, L, HIDDEN)


# speedup vs baseline: 27.0166x; 1.0263x over previous
import functools

import jax
import jax.numpy as jnp
from jax import lax
from jax.experimental import pallas as pl
from jax.experimental.pallas import tpu as pltpu
from jax.experimental.pallas import tpu_sc as plsc

B, L, RANK, HIDDEN = 4096, 200, 32, 128
N = B * L
NC, NS = 2, 16
NW = NC * NS
SEGS = 2
NSEG = N // SEGS
TOK_W = NSEG // NW
KSTR = 5
CH = KSTR * 128
GB = 64
TOK_BLK = GB * L
GRP = TOK_BLK // 4
NPS = NSEG // 4
BLKS = NSEG // TOK_BLK
CH_G = GRP // CH

_mesh = plsc.VectorSubcoreMesh(
    core_axis_name="c", subcore_axis_name="s", num_cores=NC, num_subcores=NS
)


@functools.partial(
    pl.kernel,
    out_type=jax.ShapeDtypeStruct((NPS, HIDDEN), jnp.float32),
    mesh=_mesh,
    scratch_types=[
        pltpu.VMEM((KSTR, 128), jnp.int32),
        pltpu.VMEM((CH, RANK), jnp.float32),
        pltpu.SemaphoreType.DMA,
    ],
    compiler_params=pltpu.CompilerParams(use_tc_tiling_on_sc=False),
)
def _sc_gather(ids_hbm, table_hbm, p_hbm, idx_v, rows_v, sem):
    c = lax.axis_index("c")
    s = lax.axis_index("s")
    wid = s * NC + c
    base = wid * TOK_W
    for q in range(4):
        def step(i, _, q=q):
            tok0 = base + q * GRP + i * CH
            pltpu.sync_copy(ids_hbm.at[pl.ds(tok0 // 128, KSTR)], idx_v)
            cps = [
                pltpu.async_copy(
                    table_hbm.at[idx_v.at[j]],
                    rows_v.at[pl.ds(j * 128, 128)],
                    sem,
                )
                for j in range(KSTR)
            ]
            for cp in cps:
                cp.wait()
            row0 = wid * GRP + i * CH
            pltpu.sync_copy(
                rows_v,
                p_hbm.at[pl.ds(row0, CH), pl.ds(32 * q, 32)],
            )
            return 0
        lax.fori_loop(0, CH_G, step, 0)


def _tc_body(p_ref, w2_ref, posb_ref, out_ref):
    p = p_ref[...]
    w2 = w2_ref[...]
    pos = jnp.broadcast_to(
        posb_ref[...][None], (GRP // L, L, HIDDEN)
    ).reshape(GRP, HIDDEN)
    for q in range(4):
        xq = p[:, 32 * q:32 * (q + 1)]
        yq = lax.dot_general(
            xq, w2, (((1,), (1,)), ((), ())),
            preferred_element_type=jnp.float32,
        )
        out_ref[pl.ds(q * GRP, GRP), :] = yq + pos


def _tc_body_chain(p_ref, w2_ref, posb_ref, prev_ref, out_ref):
    del prev_ref
    _tc_body(p_ref, w2_ref, posb_ref, out_ref)


def _make_tc(seg):
    in_specs = [
        pl.BlockSpec((GRP, HIDDEN), lambda g: (g, 0)),
        pl.BlockSpec((HIDDEN, RANK), lambda g: (0, 0)),
        pl.BlockSpec((L, HIDDEN), lambda g: (0, 0)),
    ]
    body = _tc_body
    kwargs = {}
    if seg > 0:
        in_specs.append(pl.BlockSpec(memory_space=pl.ANY))
        body = _tc_body_chain
        kwargs["input_output_aliases"] = {3: 0}
    return pl.pallas_call(
        body,
        grid=(BLKS,),
        in_specs=in_specs,
        out_specs=pl.BlockSpec(
            (TOK_BLK, HIDDEN), lambda g, seg=seg: (seg * BLKS + g, 0)
        ),
        out_shape=jax.ShapeDtypeStruct((N, HIDDEN), jnp.float32),
        **kwargs,
    )


_tc_calls = [_make_tc(s) for s in range(SEGS)]


def kernel(input_ids, E1, W2, b2, pos_emb):
    ids = input_ids.reshape(N // 128, 128).astype(jnp.int32)
    posb = pos_emb[0, :L] + b2[None, :]
    out = None
    for s in range(SEGS):
        ids_s = lax.slice_in_dim(ids, s * (NSEG // 128), (s + 1) * (NSEG // 128))
        p_s = _sc_gather(ids_s, E1)
        args = (p_s, W2, posb) if s == 0 else (p_s, W2, posb, out)
        out = _tc_calls[s](*args)
    return out.reshape(B, L, HIDDEN)


# speedup vs baseline: 31.2469x; 1.1566x over previous
import functools

import jax
import jax.numpy as jnp
from jax import lax
from jax.experimental import pallas as pl
from jax.experimental.pallas import tpu as pltpu
from jax.experimental.pallas import tpu_sc as plsc

B, L, RANK, HIDDEN = 4096, 200, 32, 128
N = B * L
NC, NS = 2, 16
NW = NC * NS
SEGS = 2
NSEG = N // SEGS
TOK_W = NSEG // NW
KSTR = 5
CH = KSTR * 128
GB = 64
TOK_BLK = GB * L
GRP = TOK_BLK // 4
NPS = NSEG // 4
BLKS = NSEG // TOK_BLK
CH_G = GRP // CH

_mesh = plsc.VectorSubcoreMesh(
    core_axis_name="c", subcore_axis_name="s", num_cores=NC, num_subcores=NS
)


@functools.partial(
    pl.kernel,
    out_type=jax.ShapeDtypeStruct((NPS, HIDDEN), jnp.float32),
    mesh=_mesh,
    scratch_types=[
        pltpu.VMEM((KSTR, 128), jnp.int32),
        pltpu.VMEM((CH, RANK), jnp.float32),
        pltpu.SemaphoreType.DMA,
    ],
    compiler_params=pltpu.CompilerParams(use_tc_tiling_on_sc=False),
)
def _sc_gather(ids_hbm, table_hbm, p_hbm, idx_v, rows_v, sem):
    c = lax.axis_index("c")
    s = lax.axis_index("s")
    wid = s * NC + c
    base = wid * TOK_W
    for q in range(4):
        def step(i, _, q=q):
            tok0 = base + q * GRP + i * CH
            pltpu.sync_copy(ids_hbm.at[pl.ds(tok0 // 128, KSTR)], idx_v)
            cps = [
                pltpu.async_copy(
                    table_hbm.at[idx_v.at[j]],
                    rows_v.at[pl.ds(j * 128, 128)],
                    sem,
                )
                for j in range(KSTR)
            ]
            for cp in cps:
                cp.wait()
            row0 = wid * GRP + i * CH
            pltpu.sync_copy(
                rows_v,
                p_hbm.at[pl.ds(row0, CH), pl.ds(32 * q, 32)],
            )
            return 0
        lax.fori_loop(0, CH_G, step, 0)


VOCAB = 1000000
VCH = 8192
VBLK = (VOCAB + VCH - 1) // VCH


def _repack_body(e1t_ref, out_ref):
    t = lax.dot_general(
        e1t_ref[...], jnp.eye(RANK, dtype=jnp.float32),
        (((0,), (0,)), ((), ())),
        preferred_element_type=jnp.float32,
    )
    t3 = t.reshape(VCH // 4, 4, RANK)
    for q in range(4):
        out_ref[:, 32 * q:32 * (q + 1)] = t3[:, q, :]


_tc_repack = pl.pallas_call(
    _repack_body,
    grid=(VBLK,),
    in_specs=[pl.BlockSpec((RANK, VCH), lambda g: (0, g))],
    out_specs=pl.BlockSpec((VCH // 4, HIDDEN), lambda g: (g, 0)),
    out_shape=jax.ShapeDtypeStruct((250000, HIDDEN), jnp.float32),
)


def _tc_body(p_ref, w2_ref, posb_ref, out_ref):
    p = p_ref[...]
    w2 = w2_ref[...]
    pos = jnp.broadcast_to(
        posb_ref[...][None], (GRP // L, L, HIDDEN)
    ).reshape(GRP, HIDDEN)
    for q in range(4):
        xq = p[:, 32 * q:32 * (q + 1)]
        yq = lax.dot_general(
            xq, w2, (((1,), (1,)), ((), ())),
            preferred_element_type=jnp.float32,
        )
        out_ref[pl.ds(q * GRP, GRP), :] = yq + pos


def _tc_body_chain(p_ref, w2_ref, posb_ref, prev_ref, out_ref):
    del prev_ref
    _tc_body(p_ref, w2_ref, posb_ref, out_ref)


def _make_tc(seg):
    in_specs = [
        pl.BlockSpec((GRP, HIDDEN), lambda g: (g, 0)),
        pl.BlockSpec((HIDDEN, RANK), lambda g: (0, 0)),
        pl.BlockSpec((L, HIDDEN), lambda g: (0, 0)),
    ]
    body = _tc_body
    kwargs = {}
    if seg > 0:
        in_specs.append(pl.BlockSpec(memory_space=pl.ANY))
        body = _tc_body_chain
        kwargs["input_output_aliases"] = {3: 0}
    return pl.pallas_call(
        body,
        grid=(BLKS,),
        in_specs=in_specs,
        out_specs=pl.BlockSpec(
            (TOK_BLK, HIDDEN), lambda g, seg=seg: (seg * BLKS + g, 0)
        ),
        out_shape=jax.ShapeDtypeStruct((N, HIDDEN), jnp.float32),
        **kwargs,
    )


_tc_calls = [_make_tc(s) for s in range(SEGS)]


def kernel(input_ids, E1, W2, b2, pos_emb):
    ids = input_ids.reshape(N // 128, 128).astype(jnp.int32)
    e1p = _tc_repack(E1.T)
    e1lin = e1p.reshape(1000000, RANK)
    posb = pos_emb[0, :L] + b2[None, :]
    out = None
    for s in range(SEGS):
        ids_s = lax.slice_in_dim(ids, s * (NSEG // 128), (s + 1) * (NSEG // 128))
        p_s = _sc_gather(ids_s, e1lin)
        args = (p_s, W2, posb) if s == 0 else (p_s, W2, posb, out)
        out = _tc_calls[s](*args)
    return out.reshape(B, L, HIDDEN)


# speedup vs baseline: 32.3533x; 1.0354x over previous
import functools

import jax
import jax.numpy as jnp
from jax import lax
from jax.experimental import pallas as pl
from jax.experimental.pallas import tpu as pltpu
from jax.experimental.pallas import tpu_sc as plsc

B, L, RANK, HIDDEN = 4096, 200, 32, 128
N = B * L
NC, NS = 2, 16
NW = NC * NS
SEGS = 2
NSEG = N // SEGS
TOK_W = NSEG // NW
KSTR = 5
CH = KSTR * 128
GB = 64
TOK_BLK = GB * L
GRP = TOK_BLK // 4
NPS = NSEG // 4
BLKS = NSEG // TOK_BLK
CH_G = GRP // CH

_mesh = plsc.VectorSubcoreMesh(
    core_axis_name="c", subcore_axis_name="s", num_cores=NC, num_subcores=NS
)


@functools.partial(
    pl.kernel,
    out_type=jax.ShapeDtypeStruct((NPS, HIDDEN), jnp.float32),
    mesh=_mesh,
    scratch_types=[
        pltpu.VMEM((KSTR, 128), jnp.int32),
        pltpu.VMEM((CH, RANK), jnp.float32),
        pltpu.SemaphoreType.DMA,
    ],
    compiler_params=pltpu.CompilerParams(use_tc_tiling_on_sc=False),
)
def _sc_gather(ids_hbm, table_hbm, p_hbm, idx_v, rows_v, sem):
    c = lax.axis_index("c")
    s = lax.axis_index("s")
    wid = s * NC + c
    base = wid * TOK_W
    for q in range(4):
        def step(i, _, q=q):
            tok0 = base + q * GRP + i * CH
            pltpu.sync_copy(ids_hbm.at[pl.ds(tok0 // 128, KSTR)], idx_v)
            cps = [
                pltpu.async_copy(
                    table_hbm.at[idx_v.at[j]],
                    rows_v.at[pl.ds(j * 128, 128)],
                    sem,
                )
                for j in range(KSTR)
            ]
            for cp in cps:
                cp.wait()
            row0 = wid * GRP + i * CH
            pltpu.sync_copy(
                rows_v,
                p_hbm.at[pl.ds(row0, CH), pl.ds(32 * q, 32)],
            )
            return 0
        lax.fori_loop(0, CH_G, step, 0)


VOCAB = 1000000
VCH = 8192
VBLK = (VOCAB + VCH - 1) // VCH


def _repack_body(e1t_ref, out_ref):
    t = jnp.transpose(e1t_ref[...])
    t3 = t.reshape(VCH // 4, 4, RANK)
    for q in range(4):
        out_ref[:, 32 * q:32 * (q + 1)] = t3[:, q, :]


_tc_repack = pl.pallas_call(
    _repack_body,
    grid=(VBLK,),
    in_specs=[pl.BlockSpec((RANK, VCH), lambda g: (0, g))],
    out_specs=pl.BlockSpec((VCH // 4, HIDDEN), lambda g: (g, 0)),
    out_shape=jax.ShapeDtypeStruct((250000, HIDDEN), jnp.float32),
)


def _tc_body(p_ref, w2_ref, posb_ref, out_ref):
    p = p_ref[...]
    w2 = w2_ref[...]
    pos = jnp.broadcast_to(
        posb_ref[...][None], (GRP // L, L, HIDDEN)
    ).reshape(GRP, HIDDEN)
    for q in range(4):
        xq = p[:, 32 * q:32 * (q + 1)]
        yq = lax.dot_general(
            xq, w2, (((1,), (1,)), ((), ())),
            preferred_element_type=jnp.float32,
        )
        out_ref[pl.ds(q * GRP, GRP), :] = yq + pos


def _tc_body_chain(p_ref, w2_ref, posb_ref, prev_ref, out_ref):
    del prev_ref
    _tc_body(p_ref, w2_ref, posb_ref, out_ref)


def _make_tc(seg):
    in_specs = [
        pl.BlockSpec((GRP, HIDDEN), lambda g: (g, 0)),
        pl.BlockSpec((HIDDEN, RANK), lambda g: (0, 0)),
        pl.BlockSpec((L, HIDDEN), lambda g: (0, 0)),
    ]
    body = _tc_body
    kwargs = {}
    if seg > 0:
        in_specs.append(pl.BlockSpec(memory_space=pl.ANY))
        body = _tc_body_chain
        kwargs["input_output_aliases"] = {3: 0}
    return pl.pallas_call(
        body,
        grid=(BLKS,),
        in_specs=in_specs,
        out_specs=pl.BlockSpec(
            (TOK_BLK, HIDDEN), lambda g, seg=seg: (seg * BLKS + g, 0)
        ),
        out_shape=jax.ShapeDtypeStruct((N, HIDDEN), jnp.float32),
        **kwargs,
    )


_tc_calls = [_make_tc(s) for s in range(SEGS)]


def kernel(input_ids, E1, W2, b2, pos_emb):
    ids = input_ids.reshape(N // 128, 128).astype(jnp.int32)
    e1p = _tc_repack(E1.T)
    e1lin = e1p.reshape(1000000, RANK)
    posb = pos_emb[0, :L] + b2[None, :]
    out = None
    for s in range(SEGS):
        ids_s = lax.slice_in_dim(ids, s * (NSEG // 128), (s + 1) * (NSEG // 128))
        p_s = _sc_gather(ids_s, e1lin)
        args = (p_s, W2, posb) if s == 0 else (p_s, W2, posb, out)
        out = _tc_calls[s](*args)
    return out.reshape(B, L, HIDDEN)


# speedup vs baseline: 33.5517x; 1.0370x over previous
import functools

import jax
import jax.numpy as jnp
from jax import lax
from jax.experimental import pallas as pl
from jax.experimental.pallas import tpu as pltpu
from jax.experimental.pallas import tpu_sc as plsc

B, L, RANK, HIDDEN = 4096, 200, 32, 128
N = B * L
NC, NS = 2, 16
NW = NC * NS
SEGS = 4
NSEG = N // SEGS
TOK_W = NSEG // NW
KSTR = 5
CH = KSTR * 128
GB = 64
TOK_BLK = GB * L
GRP = TOK_BLK // 4
NPS = NSEG // 4
BLKS = NSEG // TOK_BLK
CH_G = GRP // CH

_mesh = plsc.VectorSubcoreMesh(
    core_axis_name="c", subcore_axis_name="s", num_cores=NC, num_subcores=NS
)


@functools.partial(
    pl.kernel,
    out_type=jax.ShapeDtypeStruct((NPS, HIDDEN), jnp.float32),
    mesh=_mesh,
    scratch_types=[
        pltpu.VMEM((KSTR, 128), jnp.int32),
        pltpu.VMEM((CH, RANK), jnp.float32),
        pltpu.SemaphoreType.DMA,
    ],
    compiler_params=pltpu.CompilerParams(use_tc_tiling_on_sc=False),
)
def _sc_gather(ids_hbm, table_hbm, p_hbm, idx_v, rows_v, sem):
    c = lax.axis_index("c")
    s = lax.axis_index("s")
    wid = s * NC + c
    base = wid * TOK_W
    blk = wid // 2
    qh = 2 * (wid % 2)
    for q2 in range(TOK_W // GRP):
        def step(i, _, q2=q2):
            tok0 = base + q2 * GRP + i * CH
            pltpu.sync_copy(ids_hbm.at[pl.ds(tok0 // 128, KSTR)], idx_v)
            cps = [
                pltpu.async_copy(
                    table_hbm.at[idx_v.at[j]],
                    rows_v.at[pl.ds(j * 128, 128)],
                    sem,
                )
                for j in range(KSTR)
            ]
            for cp in cps:
                cp.wait()
            row0 = blk * GRP + i * CH
            pltpu.sync_copy(
                rows_v,
                p_hbm.at[pl.ds(row0, CH), pl.ds(32 * (qh + q2), 32)],
            )
            return 0
        lax.fori_loop(0, CH_G, step, 0)


VOCAB = 1000000
VCH = 16384
VBLK = (VOCAB + VCH - 1) // VCH


def _repack_body(e1t_ref, out_ref):
    t = jnp.transpose(e1t_ref[...])
    t3 = t.reshape(VCH // 4, 4, RANK)
    for q in range(4):
        out_ref[:, 32 * q:32 * (q + 1)] = t3[:, q, :]


_tc_repack = pl.pallas_call(
    _repack_body,
    grid=(VBLK,),
    in_specs=[pl.BlockSpec((RANK, VCH), lambda g: (0, g))],
    out_specs=pl.BlockSpec((VCH // 4, HIDDEN), lambda g: (g, 0)),
    out_shape=jax.ShapeDtypeStruct((250000, HIDDEN), jnp.float32),
    compiler_params=pltpu.CompilerParams(fuse_transposed_lhs_in_matmul=True),
)


def _tc_body(p_ref, w2_ref, posb_ref, out_ref):
    p = p_ref[...]
    w2 = w2_ref[...]
    pos = jnp.broadcast_to(
        posb_ref[...][None], (GRP // L, L, HIDDEN)
    ).reshape(GRP, HIDDEN)
    for q in range(4):
        xq = p[:, 32 * q:32 * (q + 1)]
        yq = lax.dot_general(
            xq, w2, (((1,), (1,)), ((), ())),
            preferred_element_type=jnp.float32,
        )
        out_ref[pl.ds(q * GRP, GRP), :] = yq + pos


def _tc_body_chain(p_ref, w2_ref, posb_ref, prev_ref, out_ref):
    del prev_ref
    _tc_body(p_ref, w2_ref, posb_ref, out_ref)


def _make_tc(seg):
    in_specs = [
        pl.BlockSpec((GRP, HIDDEN), lambda g: (g, 0)),
        pl.BlockSpec((HIDDEN, RANK), lambda g: (0, 0)),
        pl.BlockSpec((L, HIDDEN), lambda g: (0, 0)),
    ]
    body = _tc_body
    kwargs = {}
    if seg > 0:
        in_specs.append(pl.BlockSpec(memory_space=pl.ANY))
        body = _tc_body_chain
        kwargs["input_output_aliases"] = {3: 0}
    return pl.pallas_call(
        body,
        grid=(BLKS,),
        in_specs=in_specs,
        out_specs=pl.BlockSpec(
            (TOK_BLK, HIDDEN), lambda g, seg=seg: (seg * BLKS + g, 0)
        ),
        out_shape=jax.ShapeDtypeStruct((N, HIDDEN), jnp.float32),
        **kwargs,
    )


_tc_calls = [_make_tc(s) for s in range(SEGS)]


def kernel(input_ids, E1, W2, b2, pos_emb):
    ids = input_ids.reshape(N // 128, 128).astype(jnp.int32)
    e1p = _tc_repack(E1.T)
    e1lin = e1p.reshape(1000000, RANK)
    posb = pos_emb[0, :L] + b2[None, :]
    out = None
    for s in range(SEGS):
        ids_s = lax.slice_in_dim(ids, s * (NSEG // 128), (s + 1) * (NSEG // 128))
        p_s = _sc_gather(ids_s, e1lin)
        args = (p_s, W2, posb) if s == 0 else (p_s, W2, posb, out)
        out = _tc_calls[s](*args)
    return out.reshape(B, L, HIDDEN)
